# baseline (speedup 1.0000x reference)
import functools
import math

import jax
import jax.numpy as jnp
from jax import lax
from jax.experimental import pallas as pl
from jax.experimental.pallas import tpu as pltpu
from jax.experimental.pallas import tpu_sc as plsc

D = 128
SCALE = math.sqrt(float(D))

NC = 2
NS = 16
NW = NC * NS
SPC = 2
SSTRIDE = 64
CPAD = SPC * SSTRIDE
NBUF = 4


def _scale_body(t_ref, o_ref):
    o_ref[...] = t_ref[...] * SCALE


def _scale_table(table):
    rows = table.shape[0]
    blk = 2000
    return pl.pallas_call(
        _scale_body,
        grid=(rows // blk,),
        in_specs=[pl.BlockSpec((blk, D), lambda i: (i, 0))],
        out_specs=pl.BlockSpec((blk, D), lambda i: (i, 0)),
        out_shape=jax.ShapeDtypeStruct((rows, D), jnp.float32),
    )(table)


def _gather_body(nchunks, table_hbm, ids_hbm, out_hbm, idx_v, *scr):
    bufs = scr[:NBUF]
    gsems = scr[NBUF:2 * NBUF]
    wsems = scr[2 * NBUF:3 * NBUF]
    wid = lax.axis_index("s") * NC + lax.axis_index("c")
    pltpu.sync_copy(ids_hbm.at[wid], idx_v)
    pair_base = wid * nchunks

    def start_gather(c, b):
        pltpu.async_copy(table_hbm.at[idx_v.at[c]], bufs[b], gsems[b])

    for b in range(NBUF):
        start_gather(b, b)

    def step(g, issue_next):
        c0 = g * NBUF
        for b in range(NBUF):
            pltpu.make_async_copy(
                table_hbm.at[idx_v.at[0]], bufs[b], gsems[b]).wait()
            pltpu.async_copy(
                bufs[b],
                out_hbm.at[pl.ds((pair_base + c0 + b) * CPAD, CPAD)],
                wsems[b])
        for b in range(NBUF):
            pltpu.make_async_copy(
                bufs[b], out_hbm.at[pl.ds(0, CPAD)], wsems[b]).wait()
            if issue_next:
                start_gather(c0 + NBUF + b, b)

    def body(g, carry):
        step(g, True)
        return carry

    lax.fori_loop(0, nchunks // NBUF - 1, body, 0)
    step(nchunks // NBUF - 1, False)


def kernel(input_ids, table):
    nsent, seq = input_ids.shape
    npair = nsent // SPC
    nchunks = npair // NW
    ids = input_ids.reshape(npair, SPC, seq)
    ids = jnp.pad(ids, ((0, 0), (0, 0), (0, SSTRIDE - seq)))
    ids = ids.reshape(NW, nchunks, CPAD)

    scaled = _scale_table(table)

    mesh = plsc.VectorSubcoreMesh(core_axis_name="c", subcore_axis_name="s")
    gather = pl.kernel(
        functools.partial(_gather_body, nchunks),
        mesh=mesh,
        out_type=jax.ShapeDtypeStruct((nsent * SSTRIDE, D), jnp.float32),
        scratch_types=(
            [pltpu.VMEM((nchunks, CPAD), jnp.int32)]
            + [pltpu.VMEM((CPAD, D), jnp.float32) for _ in range(NBUF)]
            + [pltpu.SemaphoreType.DMA for _ in range(2 * NBUF)]
        ),
    )
    flat = gather(scaled, ids)
    return flat.reshape(nsent, SSTRIDE, D)[:, :seq, :]


# speedup vs baseline: 7.9902x; 7.9902x over previous
import functools
import math

import jax
import jax.numpy as jnp
from jax import lax
from jax.experimental import pallas as pl
from jax.experimental.pallas import tpu as pltpu
from jax.experimental.pallas import tpu_sc as plsc

D = 128
SCALE = math.sqrt(float(D))

NC = 2
NS = 16
NW = NC * NS
SPC = 2
SSTRIDE = 64
CPAD = SPC * SSTRIDE
NBUF = 4


def _scale_body(t_ref, o_ref):
    o_ref[...] = t_ref[...] * SCALE


def _scale_table(table):
    rows = table.shape[0]
    blk = 2000
    return pl.pallas_call(
        _scale_body,
        grid=(rows // blk,),
        in_specs=[pl.BlockSpec((blk, D), lambda i: (i, 0))],
        out_specs=pl.BlockSpec((blk, D), lambda i: (i, 0)),
        out_shape=jax.ShapeDtypeStruct((rows, D), jnp.float32),
    )(table)


def _gather_body(nchunks, table_hbm, ids_hbm, out_hbm, idx_v, *scr):
    bufs = scr[:NBUF]
    gsems = scr[NBUF:2 * NBUF]
    wsems = scr[2 * NBUF:3 * NBUF]
    wid = lax.axis_index("s") * NC + lax.axis_index("c")
    pltpu.sync_copy(ids_hbm.at[wid], idx_v)
    pair_base = wid * nchunks

    def start_gather(c, b):
        pltpu.async_copy(table_hbm.at[idx_v.at[c]], bufs[b], gsems[b])

    for b in range(NBUF):
        start_gather(b, b)

    def step(g, issue_next):
        c0 = g * NBUF
        for b in range(NBUF):
            pltpu.make_async_copy(
                table_hbm.at[idx_v.at[0]], bufs[b], gsems[b]).wait()
            pltpu.async_copy(
                bufs[b],
                out_hbm.at[pl.ds((pair_base + c0 + b) * CPAD, CPAD)],
                wsems[b])
        for b in range(NBUF):
            pltpu.make_async_copy(
                bufs[b], out_hbm.at[pl.ds(0, CPAD)], wsems[b]).wait()
            if issue_next:
                start_gather(c0 + NBUF + b, b)

    def body(g, carry):
        step(g, True)
        return carry

    lax.fori_loop(0, nchunks // NBUF - 1, body, 0)
    step(nchunks // NBUF - 1, False)


def kernel(input_ids, table):
    nsent, seq = input_ids.shape
    npair = nsent // SPC
    nchunks = npair // NW
    ids = input_ids.reshape(npair, SPC, seq)
    ids = jnp.concatenate([ids, ids[:, :, :SSTRIDE - seq]], axis=2)
    ids = ids.reshape(NW, nchunks, CPAD)

    scaled = _scale_table(table)

    mesh = plsc.VectorSubcoreMesh(core_axis_name="c", subcore_axis_name="s")
    gather = pl.kernel(
        functools.partial(_gather_body, nchunks),
        mesh=mesh,
        out_type=jax.ShapeDtypeStruct((nsent * SSTRIDE, D), jnp.float32),
        scratch_types=(
            [pltpu.VMEM((nchunks, CPAD), jnp.int32)]
            + [pltpu.VMEM((CPAD, D), jnp.float32) for _ in range(NBUF)]
            + [pltpu.SemaphoreType.DMA for _ in range(2 * NBUF)]
        ),
    )
    flat = gather(scaled, ids)
    return flat.reshape(nsent, SSTRIDE, D)[:, :seq, :]


# speedup vs baseline: 18.2457x; 2.2835x over previous
import functools
import math

import jax
import jax.numpy as jnp
from jax import lax
from jax.experimental import pallas as pl
from jax.experimental.pallas import tpu as pltpu
from jax.experimental.pallas import tpu_sc as plsc

D = 128
SCALE = math.sqrt(float(D))

NC = 2
NS = 16
NW = NC * NS
C = 128
NBUF = 5


def _scale_body(t_ref, o_ref):
    o_ref[...] = t_ref[...] * SCALE


def _scale_table(table):
    rows = table.shape[0]
    blk = 2000
    return pl.pallas_call(
        _scale_body,
        grid=(rows // blk,),
        in_specs=[pl.BlockSpec((blk, D), lambda i: (i, 0))],
        out_specs=pl.BlockSpec((blk, D), lambda i: (i, 0)),
        out_shape=jax.ShapeDtypeStruct((rows, D), jnp.float32),
    )(table)


def _gather_body(nchunks, b_per_w, table_hbm, ids_hbm, out_hbm,
                 idx_v, *scr):
    bufs = scr[:NBUF]
    gsems = scr[NBUF:2 * NBUF]
    wsems = scr[2 * NBUF:3 * NBUF]
    wid = lax.axis_index("s") * NC + lax.axis_index("c")
    pltpu.sync_copy(ids_hbm.at[wid], idx_v)
    base = wid * b_per_w
    niter = nchunks // NBUF

    def start_gather(c, b):
        pltpu.async_copy(table_hbm.at[idx_v.at[c]], bufs[b], gsems[b])

    for b in range(NBUF):
        start_gather(b, b)

    def step(g, issue_next):
        c0 = g * NBUF
        for b in range(NBUF):
            pltpu.make_async_copy(
                table_hbm.at[idx_v.at[0]], bufs[b], gsems[b]).wait()
            pltpu.async_copy(
                bufs[b], out_hbm.at[pl.ds(base + (c0 + b) * C, C)], wsems[b])
        for b in range(NBUF):
            pltpu.make_async_copy(
                bufs[b], out_hbm.at[pl.ds(0, C)], wsems[b]).wait()
            if issue_next:
                start_gather(c0 + NBUF + b, b)

    def body(g, carry):
        step(g, True)
        return carry

    lax.fori_loop(0, niter - 1, body, 0)
    step(niter - 1, False)


def kernel(input_ids, table):
    nsent, seq = input_ids.shape
    b_total = input_ids.size
    b_per_w = b_total // NW
    nchunks = b_per_w // C
    ids = jnp.transpose(input_ids).reshape(NW, nchunks, C)

    scaled = _scale_table(table)

    mesh = plsc.VectorSubcoreMesh(core_axis_name="c", subcore_axis_name="s")
    gather = pl.kernel(
        functools.partial(_gather_body, nchunks, b_per_w),
        mesh=mesh,
        out_type=jax.ShapeDtypeStruct((b_total, D), jnp.float32),
        scratch_types=(
            [pltpu.VMEM((nchunks, C), jnp.int32)]
            + [pltpu.VMEM((C, D), jnp.float32) for _ in range(NBUF)]
            + [pltpu.SemaphoreType.DMA for _ in range(2 * NBUF)]
        ),
    )
    flat = gather(scaled, ids)
    return jnp.transpose(flat.reshape(seq, nsent, D), (1, 0, 2))


# speedup vs baseline: 28.0924x; 1.5397x over previous
import functools
import math

import jax
import jax.numpy as jnp
from jax import lax
from jax.experimental import pallas as pl
from jax.experimental.pallas import tpu as pltpu
from jax.experimental.pallas import tpu_sc as plsc

D = 128
SCALE = math.sqrt(float(D))

NC = 2
NS = 16
NW = NC * NS
C = 128
NBUF = 5
NL = 16


def _gather_body(nchunks, b_per_w, table_hbm, ids_hbm, out_hbm,
                 idx_v, *scr):
    bufs = scr[:NBUF]
    gsems = scr[NBUF:2 * NBUF]
    wsems = scr[2 * NBUF:3 * NBUF]
    wid = lax.axis_index("s") * NC + lax.axis_index("c")
    pltpu.sync_copy(ids_hbm.at[wid], idx_v)
    base = wid * b_per_w
    niter = nchunks // NBUF

    def start_gather(c, b):
        pltpu.async_copy(table_hbm.at[idx_v.at[c]], bufs[b], gsems[b])

    for b in range(NBUF):
        start_gather(b, b)

    def scale_buf(buf):
        def row(r, carry):
            for j in range(D // NL):
                sl = pl.ds(j * NL, NL)
                buf[r, sl] = buf[r, sl] * SCALE
            return carry
        lax.fori_loop(0, C, row, 0)

    def step(g, issue_next):
        c0 = g * NBUF
        for b in range(NBUF):
            pltpu.make_async_copy(
                table_hbm.at[idx_v.at[0]], bufs[b], gsems[b]).wait()
            scale_buf(bufs[b])
            pltpu.async_copy(
                bufs[b], out_hbm.at[pl.ds(base + (c0 + b) * C, C)], wsems[b])
        for b in range(NBUF):
            pltpu.make_async_copy(
                bufs[b], out_hbm.at[pl.ds(0, C)], wsems[b]).wait()
            if issue_next:
                start_gather(c0 + NBUF + b, b)

    def body(g, carry):
        step(g, True)
        return carry

    lax.fori_loop(0, niter - 1, body, 0)
    step(niter - 1, False)


def kernel(input_ids, table):
    nsent, seq = input_ids.shape
    b_total = input_ids.size
    b_per_w = b_total // NW
    nchunks = b_per_w // C
    ids = jnp.transpose(input_ids).reshape(NW, nchunks, C)

    mesh = plsc.VectorSubcoreMesh(core_axis_name="c", subcore_axis_name="s")
    gather = pl.kernel(
        functools.partial(_gather_body, nchunks, b_per_w),
        mesh=mesh,
        out_type=jax.ShapeDtypeStruct((b_total, D), jnp.float32),
        scratch_types=(
            [pltpu.VMEM((nchunks, C), jnp.int32)]
            + [pltpu.VMEM((C, D), jnp.float32) for _ in range(NBUF)]
            + [pltpu.SemaphoreType.DMA for _ in range(2 * NBUF)]
        ),
    )
    flat = gather(table, ids)
    return jnp.transpose(flat.reshape(seq, nsent, D), (1, 0, 2))
